# Initial kernel scaffold; baseline (speedup 1.0000x reference)
import jax
import jax.numpy as jnp
from jax.experimental import pallas as pl


def kernel(embeddings, class_prototypes):
    raise NotImplementedError("write your pallas kernel here")


# speedup vs baseline: 3.0069x; 3.0069x over previous
import functools

import jax
import jax.numpy as jnp
from jax import lax
from jax.experimental import pallas as pl
from jax.experimental.pallas import tpu as pltpu
from jax.experimental.pallas import tpu_sc as plsc

NEG = -3e38
BIG_I = 1 << 30
CHUNK = 8
TOPC = 8
PB = 4096
QB = 64
GATHER_TILE = 128
EPS = 1e-12


def _phase_a_body(n_real, emb_ref, protos_ref, conf_ref, rows_ref,
                  m_s, s_s, tv_s, ti_s):
    i = pl.program_id(0)
    nb = pl.num_programs(0)
    q = emb_ref.shape[0]
    pb = protos_ref.shape[0]
    nchunks = pb // CHUNK

    @pl.when(i == 0)
    def _():
        m_s[...] = jnp.full(m_s.shape, NEG, jnp.float32)
        s_s[...] = jnp.zeros(s_s.shape, jnp.float32)
        tv_s[...] = jnp.full(tv_s.shape, NEG, jnp.float32)
        ti_s[...] = BIG_I + lax.broadcasted_iota(jnp.int32, ti_s.shape, 0)

    s = lax.dot_general(protos_ref[...], emb_ref[...], (((1,), (1,)), ((), ())),
                        preferred_element_type=jnp.float32)
    gidx = i * pb + lax.broadcasted_iota(jnp.int32, (pb, q), 0)
    s = jnp.where(gidx < n_real, s, NEG)

    bm = jnp.max(s, axis=0, keepdims=True)
    new_m = jnp.maximum(m_s[...], bm)
    s_s[...] = (s_s[...] * jnp.exp(m_s[...] - new_m)
                + jnp.sum(jnp.exp(s - new_m), axis=0, keepdims=True))
    m_s[...] = new_m

    cm = jnp.max(s.reshape(nchunks, CHUNK, q), axis=1)
    riota = lax.broadcasted_iota(jnp.int32, (nchunks, q), 0)
    bvals, bidx = [], []
    for _t in range(TOPC):
        mv = jnp.max(cm, axis=0, keepdims=True)
        sel = jnp.where(cm == mv, riota, BIG_I)
        mi = jnp.min(sel, axis=0, keepdims=True)
        bvals.append(mv)
        bidx.append(mi + i * nchunks)
        cm = jnp.where(riota == mi, NEG, cm)

    av = jnp.concatenate([tv_s[...]] + bvals, axis=0)
    ai = jnp.concatenate([ti_s[...]] + bidx, axis=0)
    nv, ni = [], []
    for _t in range(TOPC):
        mv = jnp.max(av, axis=0, keepdims=True)
        sel = jnp.where(av == mv, ai, BIG_I)
        mi = jnp.min(sel, axis=0, keepdims=True)
        nv.append(mv)
        ni.append(mi)
        av = jnp.where(ai == mi, NEG, av)
    tv_s[...] = jnp.concatenate(nv, axis=0)
    ti_s[...] = jnp.concatenate(ni, axis=0)

    @pl.when(i == nb - 1)
    def _():
        conf_ref[...] = 1.0 / s_s[...]
        ti = ti_s[...]
        rows = (jnp.broadcast_to((ti * CHUNK)[:, None, :], (TOPC, CHUNK, q))
                + lax.broadcasted_iota(jnp.int32, (TOPC, CHUNK, q), 1))
        rows_ref[...] = rows.reshape(TOPC * CHUNK, q)


def _phase_a(emb, protos, interpret=False):
    q, d = emb.shape
    n, _ = protos.shape
    nb = (n + PB - 1) // PB
    ncand = TOPC * CHUNK
    return pl.pallas_call(
        functools.partial(_phase_a_body, n),
        grid=(nb,),
        in_specs=[
            pl.BlockSpec((q, d), lambda i: (0, 0)),
            pl.BlockSpec((PB, d), lambda i: (i, 0)),
        ],
        out_specs=[
            pl.BlockSpec((1, q), lambda i: (0, 0)),
            pl.BlockSpec((ncand, q), lambda i: (0, 0)),
        ],
        out_shape=[
            jax.ShapeDtypeStruct((1, q), jnp.float32),
            jax.ShapeDtypeStruct((ncand, q), jnp.int32),
        ],
        scratch_shapes=[
            pltpu.VMEM((1, q), jnp.float32),
            pltpu.VMEM((1, q), jnp.float32),
            pltpu.VMEM((TOPC, q), jnp.float32),
            pltpu.VMEM((TOPC, q), jnp.int32),
        ],
        interpret=interpret,
    )(emb, protos)


def _sc_gather(protos, rows_flat):
    total = rows_flat.shape[0]
    d = protos.shape[1]
    info = plsc.get_sparse_core_info()
    nw = info.num_cores * info.num_subcores
    per_w = total // nw
    ntiles = per_w // GATHER_TILE
    mesh = plsc.VectorSubcoreMesh(core_axis_name="c", subcore_axis_name="s")

    @functools.partial(
        pl.kernel, mesh=mesh,
        out_type=jax.ShapeDtypeStruct((total, d), jnp.float32),
        scratch_types=[
            pltpu.VMEM((per_w,), jnp.int32),
            pltpu.VMEM((GATHER_TILE, d), jnp.float32),
            pltpu.VMEM((GATHER_TILE, d), jnp.float32),
            pltpu.SemaphoreType.DMA,
            pltpu.SemaphoreType.DMA,
        ],
    )
    def gk(protos_hbm, rows_hbm, out_hbm, idx_v, buf0, buf1, sem0, sem1):
        wid = lax.axis_index("s") * info.num_cores + lax.axis_index("c")
        base = wid * per_w
        pltpu.sync_copy(rows_hbm.at[pl.ds(base, per_w)], idx_v)
        bufs = (buf0, buf1)
        sems = (sem0, sem1)
        prev = pltpu.async_copy(
            protos_hbm.at[idx_v.at[pl.ds(0, GATHER_TILE)]], buf0, sem0)
        for j in range(ntiles):
            nxt = None
            if j + 1 < ntiles:
                nxt = pltpu.async_copy(
                    protos_hbm.at[idx_v.at[pl.ds((j + 1) * GATHER_TILE,
                                                 GATHER_TILE)]],
                    bufs[(j + 1) % 2], sems[(j + 1) % 2])
            prev.wait()
            pltpu.sync_copy(bufs[j % 2],
                            out_hbm.at[pl.ds(base + j * GATHER_TILE,
                                             GATHER_TILE)])
            prev = nxt

    return gk(protos, rows_flat)


def _rescore_body(g_ref, e_ref, r_ref, o_ref):
    g = g_ref[...].astype(jnp.bfloat16).astype(jnp.float32)
    e = e_ref[...].astype(jnp.bfloat16).astype(jnp.float32)
    sims = jnp.sum(g * e[None], axis=2)
    cidx = r_ref[0]
    for t in range(5):
        mv = jnp.max(sims, axis=0, keepdims=True)
        sel = jnp.where(sims == mv, cidx, BIG_I)
        mi = jnp.min(sel, axis=0, keepdims=True)
        o_ref[0, pl.ds(t, 1), :] = mi
        sims = jnp.where(cidx == mi, NEG, sims)


def _rescore(gathered3, embn, rows_nbq, interpret=False):
    ncand, q, d = gathered3.shape
    nb = q // QB
    return pl.pallas_call(
        _rescore_body,
        grid=(nb,),
        in_specs=[
            pl.BlockSpec((ncand, QB, d), lambda j: (0, j, 0)),
            pl.BlockSpec((QB, d), lambda j: (j, 0)),
            pl.BlockSpec((1, ncand, QB), lambda j: (j, 0, 0)),
        ],
        out_specs=pl.BlockSpec((1, 5, QB), lambda j: (j, 0, 0)),
        out_shape=jax.ShapeDtypeStruct((nb, 5, QB), jnp.int32),
        interpret=interpret,
    )(gathered3, embn, rows_nbq)


def _l2n(x):
    n = jnp.linalg.norm(x, ord=2, axis=1, keepdims=True)
    return x / jnp.maximum(n, EPS)


def kernel(embeddings, class_prototypes):
    emb = jnp.squeeze(embeddings, axis=1)
    q, d = emb.shape
    en = _l2n(emb)
    pn = _l2n(class_prototypes)
    conf2, rows_cm = _phase_a(en, pn)
    gathered = _sc_gather(pn, rows_cm.reshape(-1))
    ncand = TOPC * CHUNK
    rows_nbq = rows_cm.reshape(ncand, q // QB, QB).transpose(1, 0, 2)
    top5 = _rescore(gathered.reshape(ncand, q, d), en, rows_nbq)
    return (top5.transpose(0, 2, 1).reshape(q, 5), conf2.reshape(q))


# speedup vs baseline: 5.2953x; 1.7610x over previous
import functools

import jax
import jax.numpy as jnp
from jax import lax
from jax.experimental import pallas as pl
from jax.experimental.pallas import tpu as pltpu
from jax.experimental.pallas import tpu_sc as plsc

NEG = -3e38
BIG_I = 1 << 30
CHUNK = 16
TOPC = 5
PB = 4096
QB = 64
GATHER_TILE = 128
EPS = 1e-12


def _phase_a_body(n_real, emb_ref, protos_ref, conf_ref, rows_ref,
                  m_s, s_s, tv_s, ti_s):
    i = pl.program_id(0)
    nb = pl.num_programs(0)
    q = emb_ref.shape[0]
    pb = protos_ref.shape[0]
    nchunks = pb // CHUNK

    @pl.when(i == 0)
    def _():
        m_s[...] = jnp.full(m_s.shape, NEG, jnp.float32)
        s_s[...] = jnp.zeros(s_s.shape, jnp.float32)
        tv_s[...] = jnp.full(tv_s.shape, NEG, jnp.float32)
        ti_s[...] = BIG_I + lax.broadcasted_iota(jnp.int32, ti_s.shape, 0)

    s = lax.dot_general(protos_ref[...], emb_ref[...], (((1,), (1,)), ((), ())),
                        preferred_element_type=jnp.float32)

    cm = jnp.max(s.reshape(nchunks, CHUNK, q), axis=1)
    riota = lax.broadcasted_iota(jnp.int32, (nchunks, q), 0)
    n_chunk_real = n_real // CHUNK
    cm = jnp.where(riota + i * nchunks < n_chunk_real, cm, NEG)

    bm = jnp.max(cm, axis=0, keepdims=True)
    new_m = jnp.maximum(m_s[...], bm)
    scale = jnp.exp(m_s[...] - new_m)

    @pl.when(i < nb - 1)
    def _():
        s_s[...] = (s_s[...] * scale
                    + jnp.sum(jnp.exp(s - new_m), axis=0, keepdims=True))

    @pl.when(i == nb - 1)
    def _():
        gidx = i * pb + lax.broadcasted_iota(jnp.int32, (pb, q), 0)
        sm = jnp.where(gidx < n_real, s - new_m, NEG)
        s_s[...] = (s_s[...] * scale
                    + jnp.sum(jnp.exp(sm), axis=0, keepdims=True))

    m_s[...] = new_m

    bvals, bidx = [], []
    for _t in range(TOPC):
        mv = jnp.max(cm, axis=0, keepdims=True)
        sel = jnp.where(cm == mv, riota, BIG_I)
        mi = jnp.min(sel, axis=0, keepdims=True)
        bvals.append(mv)
        bidx.append(mi + i * nchunks)
        cm = jnp.where(riota == mi, NEG, cm)

    av = jnp.concatenate([tv_s[...]] + bvals, axis=0)
    ai = jnp.concatenate([ti_s[...]] + bidx, axis=0)
    nv, ni = [], []
    for _t in range(TOPC):
        mv = jnp.max(av, axis=0, keepdims=True)
        sel = jnp.where(av == mv, ai, BIG_I)
        mi = jnp.min(sel, axis=0, keepdims=True)
        nv.append(mv)
        ni.append(mi)
        av = jnp.where(ai == mi, NEG, av)
    tv_s[...] = jnp.concatenate(nv, axis=0)
    ti_s[...] = jnp.concatenate(ni, axis=0)

    @pl.when(i == nb - 1)
    def _():
        conf_ref[...] = 1.0 / s_s[...]
        ti = ti_s[...]
        rows = (jnp.broadcast_to((ti * CHUNK)[:, None, :], (TOPC, CHUNK, q))
                + lax.broadcasted_iota(jnp.int32, (TOPC, CHUNK, q), 1))
        rows_ref[...] = rows.reshape(TOPC * CHUNK, q)


def _phase_a(emb, protos, interpret=False):
    q, d = emb.shape
    n, _ = protos.shape
    assert n % CHUNK == 0 and PB % CHUNK == 0
    nb = (n + PB - 1) // PB
    ncand = TOPC * CHUNK
    return pl.pallas_call(
        functools.partial(_phase_a_body, n),
        grid=(nb,),
        in_specs=[
            pl.BlockSpec((q, d), lambda i: (0, 0)),
            pl.BlockSpec((PB, d), lambda i: (i, 0)),
        ],
        out_specs=[
            pl.BlockSpec((1, q), lambda i: (0, 0)),
            pl.BlockSpec((ncand, q), lambda i: (0, 0)),
        ],
        out_shape=[
            jax.ShapeDtypeStruct((1, q), jnp.float32),
            jax.ShapeDtypeStruct((ncand, q), jnp.int32),
        ],
        scratch_shapes=[
            pltpu.VMEM((1, q), jnp.float32),
            pltpu.VMEM((1, q), jnp.float32),
            pltpu.VMEM((TOPC, q), jnp.float32),
            pltpu.VMEM((TOPC, q), jnp.int32),
        ],
        interpret=interpret,
    )(emb, protos)


def _sc_gather(protos, rows_flat):
    total = rows_flat.shape[0]
    d = protos.shape[1]
    info = plsc.get_sparse_core_info()
    nw = info.num_cores * info.num_subcores
    per_w = total // nw
    ntiles = per_w // GATHER_TILE
    mesh = plsc.VectorSubcoreMesh(core_axis_name="c", subcore_axis_name="s")

    @functools.partial(
        pl.kernel, mesh=mesh,
        out_type=jax.ShapeDtypeStruct((total, d), jnp.float32),
        scratch_types=[
            pltpu.VMEM((per_w,), jnp.int32),
            pltpu.VMEM((GATHER_TILE, d), jnp.float32),
            pltpu.VMEM((GATHER_TILE, d), jnp.float32),
            pltpu.SemaphoreType.DMA,
            pltpu.SemaphoreType.DMA,
        ],
    )
    def gk(protos_hbm, rows_hbm, out_hbm, idx_v, buf0, buf1, sem0, sem1):
        wid = lax.axis_index("s") * info.num_cores + lax.axis_index("c")
        base = wid * per_w
        pltpu.sync_copy(rows_hbm.at[pl.ds(base, per_w)], idx_v)
        bufs = (buf0, buf1)
        sems = (sem0, sem1)
        prev = pltpu.async_copy(
            protos_hbm.at[idx_v.at[pl.ds(0, GATHER_TILE)]], buf0, sem0)
        for j in range(ntiles):
            nxt = None
            if j + 1 < ntiles:
                nxt = pltpu.async_copy(
                    protos_hbm.at[idx_v.at[pl.ds((j + 1) * GATHER_TILE,
                                                 GATHER_TILE)]],
                    bufs[(j + 1) % 2], sems[(j + 1) % 2])
            prev.wait()
            pltpu.sync_copy(bufs[j % 2],
                            out_hbm.at[pl.ds(base + j * GATHER_TILE,
                                             GATHER_TILE)])
            prev = nxt

    return gk(protos, rows_flat)


def _rescore_body(g_ref, e_ref, r_ref, o_ref):
    g = g_ref[...].astype(jnp.bfloat16).astype(jnp.float32)
    e = e_ref[...].astype(jnp.bfloat16).astype(jnp.float32)
    sims = jnp.sum(g * e[None], axis=2)
    cidx = r_ref[0]
    for t in range(5):
        mv = jnp.max(sims, axis=0, keepdims=True)
        sel = jnp.where(sims == mv, cidx, BIG_I)
        mi = jnp.min(sel, axis=0, keepdims=True)
        o_ref[0, pl.ds(t, 1), :] = mi
        sims = jnp.where(cidx == mi, NEG, sims)


def _rescore(gathered3, embn, rows_nbq, interpret=False):
    ncand, q, d = gathered3.shape
    nb = q // QB
    return pl.pallas_call(
        _rescore_body,
        grid=(nb,),
        in_specs=[
            pl.BlockSpec((ncand, QB, d), lambda j: (0, j, 0)),
            pl.BlockSpec((QB, d), lambda j: (j, 0)),
            pl.BlockSpec((1, ncand, QB), lambda j: (j, 0, 0)),
        ],
        out_specs=pl.BlockSpec((1, 5, QB), lambda j: (j, 0, 0)),
        out_shape=jax.ShapeDtypeStruct((nb, 5, QB), jnp.int32),
        interpret=interpret,
    )(gathered3, embn, rows_nbq)


def _l2n(x):
    n = jnp.linalg.norm(x, ord=2, axis=1, keepdims=True)
    return x / jnp.maximum(n, EPS)


def kernel(embeddings, class_prototypes):
    emb = jnp.squeeze(embeddings, axis=1)
    q, d = emb.shape
    en = _l2n(emb)
    pn = _l2n(class_prototypes)
    conf2, rows_cm = _phase_a(en, pn)
    gathered = _sc_gather(pn, rows_cm.reshape(-1))
    ncand = TOPC * CHUNK
    rows_nbq = rows_cm.reshape(ncand, q // QB, QB).transpose(1, 0, 2)
    top5 = _rescore(gathered.reshape(ncand, q, d), en, rows_nbq)
    return (top5.transpose(0, 2, 1).reshape(q, 5), conf2.reshape(q))
